# Initial kernel scaffold; baseline (speedup 1.0000x reference)
import jax
import jax.numpy as jnp
from jax.experimental import pallas as pl


def kernel(hidden_state, embedding):
    raise NotImplementedError("write your pallas kernel here")


# speedup vs baseline: 1.2888x; 1.2888x over previous
import functools

import jax
import jax.numpy as jnp
from jax import lax
from jax.experimental import pallas as pl
from jax.experimental.pallas import tpu as pltpu
from jax.experimental.pallas import tpu_sc as plsc

_K = 8192
_D = 256
_BETA = 0.25
_N = 16384
_TB = 1024
_CHUNK = 4096
_SUB = 1024
_BIG = 2 ** 30


def _bf16_rne_f32(x):
    bits = lax.bitcast_convert_type(x, jnp.int32)
    lsb = lax.shift_right_logical(bits, 16) & jnp.int32(1)
    rounded = (bits + jnp.int32(0x7FFF) + lsb) & jnp.int32(-65536)
    return lax.bitcast_convert_type(rounded, jnp.float32)


def _tc_body(flat_ref, emb_ref, rn_ref, cn_ref, idx_ref, vwin_ref):
    c = pl.program_id(1)
    rn = rn_ref[...]
    flat = flat_ref[...]

    v = None
    w = None
    for s in range(_CHUNK // _SUB):
        emb_s = emb_ref[pl.ds(s * _SUB, _SUB), :]
        mm2 = lax.dot_general(flat, emb_s, (((1,), (1,)), ((), ())),
                              preferred_element_type=jnp.float32)
        cn_s = cn_ref[:, pl.ds(s * _SUB, _SUB)]
        d = (rn + cn_s) + mm2
        vs = jnp.min(d, axis=1, keepdims=True)
        kio = lax.broadcasted_iota(jnp.int32, (_TB, _SUB), 1) + (
            c * _CHUNK + s * _SUB)
        ws = jnp.min(jnp.where(d == vs, kio, _BIG), axis=1, keepdims=True)
        if v is None:
            v, w = vs, ws
        else:
            take = vs < v
            w = jnp.where(take, ws, w)
            v = jnp.where(take, vs, v)

    @pl.when(c == 0)
    def _():
        idx_ref[...] = w
        vwin_ref[...] = v

    @pl.when(c == 1)
    def _():
        v1 = vwin_ref[...]
        w1 = idx_ref[...]
        take2 = v < _bf16_rne_f32(v1)
        idx_ref[...] = jnp.where(take2, w, w1)
        vwin_ref[...] = jnp.where(take2, v, v1)


def _tc_argmin(flat_bf16, emb_bf16, rn2, cn2):
    return pl.pallas_call(
        _tc_body,
        grid=(_N // _TB, _K // _CHUNK),
        in_specs=[
            pl.BlockSpec((_TB, _D), lambda b, c: (b, 0)),
            pl.BlockSpec((_CHUNK, _D), lambda b, c: (c, 0)),
            pl.BlockSpec((_TB, 1), lambda b, c: (b, 0)),
            pl.BlockSpec((1, _CHUNK), lambda b, c: (0, c)),
        ],
        out_specs=[
            pl.BlockSpec((_TB, 1), lambda b, c: (b, 0)),
            pl.BlockSpec((_TB, 1), lambda b, c: (b, 0)),
        ],
        out_shape=[
            jax.ShapeDtypeStruct((_N, 1), jnp.int32),
            jax.ShapeDtypeStruct((_N, 1), jnp.float32),
        ],
    )(flat_bf16, emb_bf16, rn2, cn2)


_NC = 2
_NS = 16
_ROWS_PER_W = _N // (_NC * _NS)
_GCHUNK = 128


@functools.partial(
    pl.kernel,
    out_type=jax.ShapeDtypeStruct((_N, _D), jnp.float32),
    mesh=plsc.VectorSubcoreMesh(core_axis_name="c", subcore_axis_name="s"),
    scratch_types=[
        pltpu.VMEM((_ROWS_PER_W // _GCHUNK, _GCHUNK), jnp.int32),
        pltpu.VMEM((_GCHUNK, _D), jnp.float32),
        pltpu.SemaphoreType.DMA,
    ],
)
def _sc_gather(table_hbm, idx_hbm, out_hbm, idx_v, rows_v, sem):
    wid = lax.axis_index("s") * _NC + lax.axis_index("c")
    nchunk = _ROWS_PER_W // _GCHUNK
    base_row = wid * nchunk
    pltpu.sync_copy(idx_hbm.at[pl.ds(base_row, nchunk)], idx_v)
    for j in range(nchunk):
        pltpu.async_copy(table_hbm.at[idx_v.at[j]], rows_v, sem).wait()
        pltpu.sync_copy(rows_v,
                        out_hbm.at[pl.ds(wid * _ROWS_PER_W + j * _GCHUNK,
                                         _GCHUNK)])


def kernel(hidden_state, embedding):
    batch = hidden_state.shape[0]
    flat = jnp.transpose(hidden_state, (0, 2, 3, 1)).reshape(-1, _D)
    rn = jnp.sum(flat ** 2, axis=1)
    cn = jnp.sum(embedding ** 2, axis=1)
    flat_bf16 = flat.astype(jnp.bfloat16)
    emb_bf16 = (-2.0 * embedding).astype(jnp.bfloat16)

    flat_bf16, emb_bf16, rn, cn = lax.optimization_barrier(
        (flat_bf16, emb_bf16, rn, cn))

    idx2, vwin2 = _tc_argmin(flat_bf16, emb_bf16,
                             rn.reshape(_N, 1), cn.reshape(1, _K))
    idx = idx2[:, 0]
    loss = ((1.0 + _BETA) * jnp.sum(vwin2[:, 0]) / (_N * _D)).astype(jnp.float32)

    gathered = _sc_gather(embedding, idx.reshape(_N // _GCHUNK, _GCHUNK))
    h_quant = jnp.transpose(gathered.reshape(batch, 32, 32, _D), (0, 3, 1, 2))
    return h_quant, loss, idx.reshape(batch, -1)
